# baseline (speedup 1.0000x reference)
import functools

import jax
import jax.numpy as jnp
from jax import lax
from jax.experimental import pallas as pl
from jax.experimental.pallas import tpu as pltpu
from jax.experimental.pallas import tpu_sc as plsc

N = 10000
DIN = 128
DH = 128
DC = 64
E = 320000

NC = 2
NS = 16
NW = NC * NS

N_PAD = 10240
ZROWS = N_PAD // NS

E_ALL = E + N
CHUNK = 128
NCHUNK = 2 * (-(-E_ALL // (2 * NW * CHUNK)))
EPT = NCHUNK * CHUNK
E_PAD = EPT * NW

_MESH = plsc.VectorSubcoreMesh(core_axis_name="c", subcore_axis_name="s")


def _make_sc_scatter(D):
    @functools.partial(
        pl.kernel,
        out_type=jax.ShapeDtypeStruct((NC, N_PAD, D), jnp.float32),
        mesh=_MESH,
        compiler_params=pltpu.CompilerParams(use_tc_tiling_on_sc=False),
        scratch_types=[
            pltpu.VMEM((CHUNK,), jnp.int32),
            pltpu.VMEM((CHUNK,), jnp.int32),
            pltpu.VMEM((CHUNK, D), jnp.float32),
            pltpu.VMEM((CHUNK,), jnp.int32),
            pltpu.VMEM((CHUNK,), jnp.int32),
            pltpu.VMEM((CHUNK, D), jnp.float32),
            pltpu.VMEM_SHARED((N_PAD, D), jnp.float32),
            pltpu.SemaphoreType.DMA,
            pltpu.SemaphoreType.DMA,
        ],
    )
    def sc_scatter(g_hbm, src_hbm, dst_hbm, zeros_hbm, out_hbm,
                   src0, dst0, rows0, src1, dst1, rows1, acc_sh, sem0, sem1):
        cid = lax.axis_index("c")
        sid = lax.axis_index("s")
        wid = sid * NC + cid
        base = wid * EPT
        srcs = (src0, src1)
        dsts = (dst0, dst1)
        rows = (rows0, rows1)
        sems = (sem0, sem1)
        pltpu.sync_copy(zeros_hbm, acc_sh.at[pl.ds(sid * ZROWS, ZROWS)])
        plsc.subcore_barrier()

        pltpu.sync_copy(src_hbm.at[pl.ds(base, CHUNK)], src0)
        pltpu.sync_copy(dst_hbm.at[pl.ds(base, CHUNK)], dst0)
        pltpu.async_copy(g_hbm.at[src0], rows0, sem0)

        def body(i, carry):
            for b in range(2):
                j = 2 * i + b
                nb = 1 - b

                @pl.when(j + 1 < NCHUNK)
                def _prefetch():
                    noff = base + (j + 1) * CHUNK
                    pltpu.sync_copy(src_hbm.at[pl.ds(noff, CHUNK)], srcs[nb])
                    pltpu.sync_copy(dst_hbm.at[pl.ds(noff, CHUNK)], dsts[nb])
                    pltpu.async_copy(g_hbm.at[srcs[nb]], rows[nb], sems[nb])

                pltpu.make_async_copy(g_hbm.at[srcs[b]], rows[b], sems[b]).wait()
                pltpu.sync_copy(rows[b], acc_sh.at[dsts[b]], add=True)
            return carry

        lax.fori_loop(0, NCHUNK // 2, body, 0)
        plsc.subcore_barrier()
        pltpu.sync_copy(acc_sh.at[pl.ds(sid * ZROWS, ZROWS)],
                        out_hbm.at[cid, pl.ds(sid * ZROWS, ZROWS)])

    return sc_scatter


_sc_scatter_h = _make_sc_scatter(DH)
_sc_scatter_c = _make_sc_scatter(DC)


@functools.partial(
    pl.kernel,
    out_type=jax.ShapeDtypeStruct((NC, N_PAD), jnp.float32),
    mesh=_MESH,
    scratch_types=[
        pltpu.VMEM((CHUNK,), jnp.int32),
        pltpu.VMEM((CHUNK,), jnp.float32),
        pltpu.VMEM_SHARED((N_PAD,), jnp.float32),
    ],
)
def _sc_deg(dst_hbm, ones_hbm, zeros_hbm, out_hbm, dst_v, ones_v, acc_sh):
    cid = lax.axis_index("c")
    sid = lax.axis_index("s")
    wid = sid * NC + cid
    pltpu.sync_copy(zeros_hbm, acc_sh.at[pl.ds(sid * ZROWS, ZROWS)])
    pltpu.sync_copy(ones_hbm, ones_v)
    plsc.subcore_barrier()

    def body(j, carry):
        off = wid * EPT + j * CHUNK
        pltpu.sync_copy(dst_hbm.at[pl.ds(off, CHUNK)], dst_v)
        pltpu.sync_copy(ones_v, acc_sh.at[dst_v], add=True)
        return carry

    lax.fori_loop(0, NCHUNK, body, 0)
    plsc.subcore_barrier()
    pltpu.sync_copy(acc_sh.at[pl.ds(sid * ZROWS, ZROWS)],
                    out_hbm.at[cid, pl.ds(sid * ZROWS, ZROWS)])


_R = 1280


def _tc_a_body(x_ref, w_ref, deg_ref, g_ref, dinv_ref):
    deg = deg_ref[:, 0:1] + deg_ref[:, 1:2]
    dinv = jnp.where(deg > 0, lax.rsqrt(deg), 0.0)
    h = jnp.dot(x_ref[...], w_ref[...], preferred_element_type=jnp.float32)
    g_ref[...] = h * dinv
    dinv_ref[...] = dinv


def _tc_a(x_pad, W1, deg_pair):
    return pl.pallas_call(
        _tc_a_body,
        grid=(N_PAD // _R,),
        in_specs=[
            pl.BlockSpec((_R, DIN), lambda i: (i, 0)),
            pl.BlockSpec((DIN, DH), lambda i: (0, 0)),
            pl.BlockSpec((_R, 2), lambda i: (i, 0)),
        ],
        out_specs=[
            pl.BlockSpec((_R, DH), lambda i: (i, 0)),
            pl.BlockSpec((_R, 1), lambda i: (i, 0)),
        ],
        out_shape=[
            jax.ShapeDtypeStruct((N_PAD, DH), jnp.float32),
            jax.ShapeDtypeStruct((N_PAD, 1), jnp.float32),
        ],
    )(x_pad, W1, deg_pair)


def _tc_b_body(s_ref, dinv_ref, b_ref, w_ref, g2_ref):
    dinv = dinv_ref[...]
    z = (s_ref[0] + s_ref[1]) * dinv + b_ref[...]
    h = jnp.maximum(z, 0.0)
    h2 = jnp.dot(h, w_ref[...], preferred_element_type=jnp.float32)
    g2_ref[...] = h2 * dinv


def _tc_b(s1, dinv, b1, W3):
    return pl.pallas_call(
        _tc_b_body,
        grid=(N_PAD // _R,),
        in_specs=[
            pl.BlockSpec((NC, _R, DH), lambda i: (0, i, 0)),
            pl.BlockSpec((_R, 1), lambda i: (i, 0)),
            pl.BlockSpec((1, DH), lambda i: (0, 0)),
            pl.BlockSpec((DH, DC), lambda i: (0, 0)),
        ],
        out_specs=pl.BlockSpec((_R, DC), lambda i: (i, 0)),
        out_shape=jax.ShapeDtypeStruct((N_PAD, DC), jnp.float32),
    )(s1, dinv, b1, W3)


def _tc_c_body(s_ref, dinv_ref, b_ref, out_ref):
    out_ref[...] = (s_ref[0] + s_ref[1]) * dinv_ref[...] + b_ref[...]


def _tc_c(s2, dinv, b3):
    return pl.pallas_call(
        _tc_c_body,
        grid=(N_PAD // _R,),
        in_specs=[
            pl.BlockSpec((NC, _R, DC), lambda i: (0, i, 0)),
            pl.BlockSpec((_R, 1), lambda i: (i, 0)),
            pl.BlockSpec((1, DC), lambda i: (0, 0)),
        ],
        out_specs=pl.BlockSpec((_R, DC), lambda i: (i, 0)),
        out_shape=jax.ShapeDtypeStruct((N_PAD, DC), jnp.float32),
    )(s2, dinv, b3)


def kernel(x, edge_index, W1, b1, W3, b3):
    src = edge_index[0].astype(jnp.int32)
    dst = edge_index[1].astype(jnp.int32)
    loop = jnp.arange(N, dtype=jnp.int32)
    pad_s = jnp.zeros((E_PAD - E_ALL,), jnp.int32)
    pad_d = jnp.full((E_PAD - E_ALL,), N, jnp.int32)
    src_all = jnp.concatenate([src, loop, pad_s])
    dst_all = jnp.concatenate([dst, loop, pad_d])

    zeros_h = jnp.zeros((ZROWS, DH), jnp.float32)
    zeros_c = jnp.zeros((ZROWS, DC), jnp.float32)
    zeros_1 = jnp.zeros((ZROWS,), jnp.float32)
    ones_k = jnp.ones((CHUNK,), jnp.float32)

    deg2 = _sc_deg(dst_all, ones_k, zeros_1)
    deg_pair = deg2.T

    x_pad = jnp.zeros((N_PAD, DIN), jnp.float32).at[:N].set(x)
    g1, dinv = _tc_a(x_pad, W1, deg_pair)
    s1 = _sc_scatter_h(g1, src_all, dst_all, zeros_h)
    g2 = _tc_b(s1, dinv, b1.reshape(1, DH), W3)
    s2 = _sc_scatter_c(g2, src_all, dst_all, zeros_c)
    out = _tc_c(s2, dinv, b3.reshape(1, DC))
    return out[:N]


# speedup vs baseline: 1.0409x; 1.0409x over previous
import functools

import jax
import jax.numpy as jnp
from jax import lax
from jax.experimental import pallas as pl
from jax.experimental.pallas import tpu as pltpu
from jax.experimental.pallas import tpu_sc as plsc

N = 10000
DIN = 128
DH = 128
DC = 64
E = 320000

NC = 2
NS = 16
NW = NC * NS

N_PAD = 10240
ZROWS = N_PAD // NS

E_ALL = E + N
CHUNK = 128
NCHUNK = -(-E_ALL // (NW * CHUNK))
EPT = NCHUNK * CHUNK
E_PAD = EPT * NW

_MESH = plsc.VectorSubcoreMesh(core_axis_name="c", subcore_axis_name="s")


def _make_sc_scatter(D):
    @functools.partial(
        pl.kernel,
        out_type=jax.ShapeDtypeStruct((NC, N_PAD, D), jnp.float32),
        mesh=_MESH,
        compiler_params=pltpu.CompilerParams(use_tc_tiling_on_sc=False),
        scratch_types=[
            pltpu.VMEM((CHUNK,), jnp.int32),
            pltpu.VMEM((CHUNK,), jnp.int32),
            pltpu.VMEM((CHUNK, D), jnp.float32),
            pltpu.VMEM_SHARED((N_PAD, D), jnp.float32),
        ],
    )
    def sc_scatter(g_hbm, src_hbm, dst_hbm, zeros_hbm, out_hbm,
                   src_v, dst_v, rows_v, acc_sh):
        cid = lax.axis_index("c")
        sid = lax.axis_index("s")
        wid = sid * NC + cid
        pltpu.sync_copy(zeros_hbm, acc_sh.at[pl.ds(sid * ZROWS, ZROWS)])
        plsc.subcore_barrier()

        def body(j, carry):
            off = wid * EPT + j * CHUNK
            pltpu.sync_copy(src_hbm.at[pl.ds(off, CHUNK)], src_v)
            pltpu.sync_copy(dst_hbm.at[pl.ds(off, CHUNK)], dst_v)
            pltpu.sync_copy(g_hbm.at[src_v], rows_v)
            pltpu.sync_copy(rows_v, acc_sh.at[dst_v], add=True)
            return carry

        lax.fori_loop(0, NCHUNK, body, 0)
        plsc.subcore_barrier()
        pltpu.sync_copy(acc_sh.at[pl.ds(sid * ZROWS, ZROWS)],
                        out_hbm.at[cid, pl.ds(sid * ZROWS, ZROWS)])

    return sc_scatter


_sc_scatter_h = _make_sc_scatter(DH)
_sc_scatter_c = _make_sc_scatter(DC)


@functools.partial(
    pl.kernel,
    out_type=jax.ShapeDtypeStruct((NC, N_PAD), jnp.float32),
    mesh=_MESH,
    scratch_types=[
        pltpu.VMEM((CHUNK,), jnp.int32),
        pltpu.VMEM((CHUNK,), jnp.float32),
        pltpu.VMEM_SHARED((N_PAD,), jnp.float32),
    ],
)
def _sc_deg(dst_hbm, ones_hbm, zeros_hbm, out_hbm, dst_v, ones_v, acc_sh):
    cid = lax.axis_index("c")
    sid = lax.axis_index("s")
    wid = sid * NC + cid
    pltpu.sync_copy(zeros_hbm, acc_sh.at[pl.ds(sid * ZROWS, ZROWS)])
    pltpu.sync_copy(ones_hbm, ones_v)
    plsc.subcore_barrier()

    def body(j, carry):
        off = wid * EPT + j * CHUNK
        pltpu.sync_copy(dst_hbm.at[pl.ds(off, CHUNK)], dst_v)
        pltpu.sync_copy(ones_v, acc_sh.at[dst_v], add=True)
        return carry

    lax.fori_loop(0, NCHUNK, body, 0)
    plsc.subcore_barrier()
    pltpu.sync_copy(acc_sh.at[pl.ds(sid * ZROWS, ZROWS)],
                    out_hbm.at[cid, pl.ds(sid * ZROWS, ZROWS)])


_R = 1280


def _tc_a_body(x_ref, w_ref, deg_ref, g_ref, dinv_ref):
    deg = deg_ref[:, 0:1] + deg_ref[:, 1:2]
    dinv = jnp.where(deg > 0, lax.rsqrt(deg), 0.0)
    h = jnp.dot(x_ref[...], w_ref[...], preferred_element_type=jnp.float32)
    g_ref[...] = h * dinv
    dinv_ref[...] = dinv


def _tc_a(x_pad, W1, deg_pair):
    return pl.pallas_call(
        _tc_a_body,
        grid=(N_PAD // _R,),
        in_specs=[
            pl.BlockSpec((_R, DIN), lambda i: (i, 0)),
            pl.BlockSpec((DIN, DH), lambda i: (0, 0)),
            pl.BlockSpec((_R, 2), lambda i: (i, 0)),
        ],
        out_specs=[
            pl.BlockSpec((_R, DH), lambda i: (i, 0)),
            pl.BlockSpec((_R, 1), lambda i: (i, 0)),
        ],
        out_shape=[
            jax.ShapeDtypeStruct((N_PAD, DH), jnp.float32),
            jax.ShapeDtypeStruct((N_PAD, 1), jnp.float32),
        ],
    )(x_pad, W1, deg_pair)


def _tc_b_body(s_ref, dinv_ref, b_ref, w_ref, g2_ref):
    dinv = dinv_ref[...]
    z = (s_ref[0] + s_ref[1]) * dinv + b_ref[...]
    h = jnp.maximum(z, 0.0)
    h2 = jnp.dot(h, w_ref[...], preferred_element_type=jnp.float32)
    g2_ref[...] = h2 * dinv


def _tc_b(s1, dinv, b1, W3):
    return pl.pallas_call(
        _tc_b_body,
        grid=(N_PAD // _R,),
        in_specs=[
            pl.BlockSpec((NC, _R, DH), lambda i: (0, i, 0)),
            pl.BlockSpec((_R, 1), lambda i: (i, 0)),
            pl.BlockSpec((1, DH), lambda i: (0, 0)),
            pl.BlockSpec((DH, DC), lambda i: (0, 0)),
        ],
        out_specs=pl.BlockSpec((_R, DC), lambda i: (i, 0)),
        out_shape=jax.ShapeDtypeStruct((N_PAD, DC), jnp.float32),
    )(s1, dinv, b1, W3)


def _tc_c_body(s_ref, dinv_ref, b_ref, out_ref):
    out_ref[...] = (s_ref[0] + s_ref[1]) * dinv_ref[...] + b_ref[...]


def _tc_c(s2, dinv, b3):
    return pl.pallas_call(
        _tc_c_body,
        grid=(N_PAD // _R,),
        in_specs=[
            pl.BlockSpec((NC, _R, DC), lambda i: (0, i, 0)),
            pl.BlockSpec((_R, 1), lambda i: (i, 0)),
            pl.BlockSpec((1, DC), lambda i: (0, 0)),
        ],
        out_specs=pl.BlockSpec((_R, DC), lambda i: (i, 0)),
        out_shape=jax.ShapeDtypeStruct((N_PAD, DC), jnp.float32),
    )(s2, dinv, b3)


def kernel(x, edge_index, W1, b1, W3, b3):
    src = edge_index[0].astype(jnp.int32)
    dst = edge_index[1].astype(jnp.int32)
    loop = jnp.arange(N, dtype=jnp.int32)
    pad_s = jnp.zeros((E_PAD - E_ALL,), jnp.int32)
    pad_d = jnp.full((E_PAD - E_ALL,), N, jnp.int32)
    src_all = jnp.concatenate([src, loop, pad_s])
    dst_all = jnp.concatenate([dst, loop, pad_d])

    zeros_h = jnp.zeros((ZROWS, DH), jnp.float32)
    zeros_c = jnp.zeros((ZROWS, DC), jnp.float32)
    zeros_1 = jnp.zeros((ZROWS,), jnp.float32)
    ones_k = jnp.ones((CHUNK,), jnp.float32)

    deg2 = _sc_deg(dst_all, ones_k, zeros_1)
    deg_pair = deg2.T

    x_pad = jnp.zeros((N_PAD, DIN), jnp.float32).at[:N].set(x)
    g1, dinv = _tc_a(x_pad, W1, deg_pair)
    s1 = _sc_scatter_h(g1, src_all, dst_all, zeros_h)
    g2 = _tc_b(s1, dinv, b1.reshape(1, DH), W3)
    s2 = _sc_scatter_c(g2, src_all, dst_all, zeros_c)
    out = _tc_c(s2, dinv, b3.reshape(1, DC))
    return out[:N]
